# baseline (speedup 1.0000x reference)
import functools

import jax
import jax.numpy as jnp
from jax import lax
from jax.experimental import pallas as pl
from jax.experimental.pallas import tpu as pltpu
from jax.experimental.pallas import tpu_sc as plsc

NUM_WORKERS = 32
B_TOTAL = 4096 * 50
D = 64
B_PER_W = B_TOTAL // NUM_WORKERS
CHUNK = 128
N_CHUNKS = B_PER_W // CHUNK
SCALE_F = 8.0


def _emb_body(table, idx, out, idx_v, fbuf, obuf, gsem):
    w = lax.axis_index("s") * 2 + lax.axis_index("c")
    base = w * B_PER_W

    pltpu.sync_copy(idx.at[w], idx_v)

    ev = lax.iota(jnp.int32, 16) * 2
    cols = [(ev + 32 * h, ev + 32 * h + 1) for h in range(2)]

    def compute_row(r, _):
        rv = jnp.full((16,), r, dtype=jnp.int32)
        for h in range(2):
            ce, co = cols[h]
            a = plsc.load_gather(fbuf, [rv, ce]) * SCALE_F
            b = plsc.load_gather(fbuf, [rv, co]) * SCALE_F
            p = plsc.pack(a, b, format=plsc.PackFormat.INTERLEAVED)
            obuf[r, pl.ds(32 * h, 32)] = p
        return 0

    def chunk_body(j, _):
        pltpu.async_copy(table.at[idx_v.at[j]], fbuf, gsem).wait()
        lax.fori_loop(0, CHUNK, compute_row, 0, unroll=2)
        pltpu.sync_copy(obuf, out.at[pl.ds(base + j * CHUNK, CHUNK)])
        return 0

    lax.fori_loop(0, N_CHUNKS, chunk_body, 0)


_emb = functools.partial(
    pl.kernel,
    out_type=jax.ShapeDtypeStruct((B_TOTAL, D), jnp.bfloat16),
    mesh=plsc.VectorSubcoreMesh(core_axis_name="c", subcore_axis_name="s"),
    scratch_types=[
        pltpu.VMEM((N_CHUNKS, CHUNK), jnp.int32),
        pltpu.VMEM((CHUNK, D), jnp.float32),
        pltpu.VMEM((CHUNK, D), jnp.bfloat16),
        pltpu.SemaphoreType.DMA,
    ],
    compiler_params=pltpu.CompilerParams(
        needs_layout_passes=False,
        use_tc_tiling_on_sc=False,
    ),
)(_emb_body)


def kernel(input, weight):
    idx = jnp.reshape(input, (NUM_WORKERS, N_CHUNKS, CHUNK))
    out = _emb(weight, idx)
    return out.reshape(input.shape[0], input.shape[1], D)
